# baseline (speedup 1.0000x reference)
import functools

import jax
import jax.numpy as jnp
from jax import lax
from jax.experimental import pallas as pl
from jax.experimental.pallas import tpu as pltpu
from jax.experimental.pallas import tpu_sc as plsc

B, F, V, D = 16384, 26, 1000000, 16
EPS = 1e-5

NW = 32
IPW = B * F // NW
NG = 8
RPG = IPW // NG
NPLANE = 4
NSLOT = NPLANE * B * 128 // D


BPW = B // NW


def _sc_gather(x1flat, slotflat, blocflat, x2flat, table, w_flat):
    mesh = plsc.VectorSubcoreMesh(core_axis_name="c", subcore_axis_name="s")

    @functools.partial(
        pl.kernel,
        mesh=mesh,
        compiler_params=pltpu.CompilerParams(use_tc_tiling_on_sc=False, needs_layout_passes=False),
        out_type=(
            jax.ShapeDtypeStruct((NSLOT, D), jnp.float32),
            jax.ShapeDtypeStruct((B,), jnp.float32),
        ),
        scratch_types=(
            pltpu.VMEM((IPW,), jnp.int32),
            pltpu.VMEM((IPW,), jnp.int32),
            pltpu.VMEM((IPW,), jnp.int32),
            pltpu.VMEM((IPW,), jnp.float32),
            pltpu.VMEM((RPG, D), jnp.float32),
            pltpu.VMEM((RPG, D), jnp.float32),
            pltpu.VMEM((RPG,), jnp.float32),
            pltpu.VMEM((RPG,), jnp.float32),
            pltpu.VMEM((BPW,), jnp.float32),
            pltpu.SemaphoreType.DMA,
            pltpu.SemaphoreType.DMA,
            pltpu.SemaphoreType.DMA,
            pltpu.SemaphoreType.DMA,
            pltpu.SemaphoreType.DMA,
            pltpu.SemaphoreType.DMA,
        ),
    )
    def k(x1_hbm, slot_hbm, bloc_hbm, x2_hbm, tab_hbm, w_hbm, h_out, ws_out,
          idx, slot, bloc, x2v, buf0, buf1, wb0, wb1, acc,
          s0, s1, sw0, sw1, sc0, sc1):
        cid = lax.axis_index("c")
        sid = lax.axis_index("s")
        wid = sid * 2 + cid
        i0 = wid * IPW
        pltpu.sync_copy(x1_hbm.at[pl.ds(i0, IPW)], idx)
        pltpu.sync_copy(slot_hbm.at[pl.ds(i0, IPW)], slot)
        pltpu.sync_copy(bloc_hbm.at[pl.ds(i0, IPW)], bloc)
        pltpu.sync_copy(x2_hbm.at[pl.ds(i0, IPW)], x2v)
        for z in range(BPW // 16):
            acc[pl.ds(z * 16, 16)] = jnp.zeros((16,), jnp.float32)

        bufs = (buf0, buf1)
        sems = (s0, s1)
        wbufs = (wb0, wb1)
        wsems = (sw0, sw1)
        scsems = (sc0, sc1)

        def fire(g):
            return pltpu.async_copy(
                tab_hbm.at[idx.at[pl.ds(g * RPG, RPG)]],
                bufs[g % 2], sems[g % 2])

        def fire_w(g):
            return pltpu.async_copy(
                w_hbm.at[idx.at[pl.ds(g * RPG, RPG)]],
                wbufs[g % 2], wsems[g % 2])

        def fire_scatter(g):
            return pltpu.async_copy(
                bufs[g % 2], h_out.at[slot.at[pl.ds(g * RPG, RPG)]],
                scsems[g % 2])

        def accum_w(g):
            wb = wbufs[g % 2]

            def chunk(c, _):
                o = g * RPG + c * 16
                wo = c * 16
                prod = wb[pl.ds(wo, 16)] * x2v[pl.ds(o, 16)]
                plsc.addupdate_scatter(acc, [bloc[pl.ds(o, 16)]], prod)
                return _

            lax.fori_loop(0, RPG // 16, chunk, 0)

        hg = {0: fire(0)}
        hw = {0: fire_w(0)}
        hs = {}
        for g in range(NG):
            if g + 1 < NG:
                if g >= 1:
                    hs.pop(g - 1).wait()
                hg[g + 1] = fire(g + 1)
                hw[g + 1] = fire_w(g + 1)
            hg.pop(g).wait()
            hs[g] = fire_scatter(g)
            hw.pop(g).wait()
            accum_w(g)
        hs.pop(NG - 2).wait()
        hs.pop(NG - 1).wait()
        pltpu.sync_copy(acc, ws_out.at[pl.ds(wid * BPW, BPW)])

    return k(x1flat, slotflat, blocflat, x2flat, table, w_flat)


def _tc_forward(h4, wsum2, x2, a11, w0p, b0p, w1p, b1p, w2p, b2p, w3, b3p):
    R = 1024
    G = B // R
    H0, H1, H2 = 100, 60, 20

    def body(h_ref, w_ref, x2_ref, a_ref, w0_ref, b0_ref, w1_ref, b1_ref,
             w2_ref, b2_ref, w3_ref, b3_ref, o_ref):
        x2b = x2_ref[...]
        s = jnp.zeros((R, D), jnp.float32)
        q = jnp.zeros((R, D), jnp.float32)
        acc = jnp.zeros((R, H0), jnp.float32)
        for j in range(NPLANE):
            hj = h_ref[j]
            for fo in range(8):
                f = j * 8 + fo
                if f >= F:
                    break
                ef = hj[:, fo * D:(fo + 1) * D] * x2b[:, f:f + 1]
                s = s + ef
                q = q + ef * ef
                acc = acc + jnp.dot(ef, w0_ref[pl.ds(f * D, D), :],
                                    preferred_element_type=jnp.float32)
        fm = 0.5 * (jnp.sum(s * s, axis=1, keepdims=True)
                    - jnp.sum(q, axis=1, keepdims=True))
        wsum = w_ref[...]
        h1 = jnp.maximum(acc + b0_ref[...], 0.0)
        h2 = jnp.maximum(jnp.dot(h1, w1_ref[...],
                                 preferred_element_type=jnp.float32)
                         + b1_ref[...], 0.0)
        h3 = jnp.maximum(jnp.dot(h2, w2_ref[...],
                                 preferred_element_type=jnp.float32)
                         + b2_ref[...], 0.0)
        deep = jnp.dot(h3, w3_ref[...],
                       preferred_element_type=jnp.float32) + b3_ref[...]
        z = (wsum + fm) * a_ref[...] + deep
        o_ref[...] = jax.nn.sigmoid(z)

    return pl.pallas_call(
        body,
        grid=(G,),
        in_specs=[
            pl.BlockSpec((NPLANE, R, 128), lambda i: (0, i, 0)),
            pl.BlockSpec((R, 1), lambda i: (i, 0)),
            pl.BlockSpec((R, F), lambda i: (i, 0)),
            pl.BlockSpec((1, 1), lambda i: (0, 0)),
            pl.BlockSpec((F * D, H0), lambda i: (0, 0)),
            pl.BlockSpec((1, H0), lambda i: (0, 0)),
            pl.BlockSpec((H0, H1), lambda i: (0, 0)),
            pl.BlockSpec((1, H1), lambda i: (0, 0)),
            pl.BlockSpec((H1, H2), lambda i: (0, 0)),
            pl.BlockSpec((1, H2), lambda i: (0, 0)),
            pl.BlockSpec((H2, 1), lambda i: (0, 0)),
            pl.BlockSpec((1, 1), lambda i: (0, 0)),
        ],
        out_specs=pl.BlockSpec((R, 1), lambda i: (i, 0)),
        out_shape=jax.ShapeDtypeStruct((B, 1), jnp.float32),
    )(h4, wsum2, x2, a11, w0p, b0p, w1p, b1p, w2p, b2p, w3, b3p)


def kernel(X1, X2, embed_table, w_table, lin_w, lin_b, w0, b0, g0, bt0,
           w1, b1, g1, bt1, w2, b2, g2, bt2, w3, b3):
    x1b = X1.reshape(-1)
    kk = jnp.arange(B * F, dtype=jnp.int32)
    bb = kk // F
    ff = kk % F
    slotc = (ff // 8) * (B * 8) + bb * 8 + (ff % 8)
    bloc = bb % BPW
    x2flat = X2.reshape(-1)
    w_flat = w_table.T.reshape(-1)
    h_raw, wsum = _sc_gather(x1b, slotc, bloc, x2flat, embed_table, w_flat)
    h4 = h_raw.reshape(NPLANE, B, 128)
    wsum2 = wsum[:, None]

    inv = 1.0 / jnp.sqrt(1.0 + EPS)
    s0 = g0 * inv
    s1 = g1 * inv
    s2 = g2 * inv
    w0p = w0 * s0[None, :]
    b0p = (b0 * s0 + bt0)[None, :]
    w1p = w1 * s1[None, :]
    b1p = (b1 * s1 + bt1)[None, :]
    w2p = w2 * s2[None, :]
    b2p = (b2 * s2 + bt2)[None, :]
    b3p = (b3 + lin_b)[None, :]

    return _tc_forward(h4, wsum2, X2, lin_w, w0p, b0p, w1p, b1p,
                       w2p, b2p, w3, b3p)


# speedup vs baseline: 1.0936x; 1.0936x over previous
import functools

import jax
import jax.numpy as jnp
from jax import lax
from jax.experimental import pallas as pl
from jax.experimental.pallas import tpu as pltpu
from jax.experimental.pallas import tpu_sc as plsc

B, F, V, D = 16384, 26, 1000000, 16
EPS = 1e-5

NW = 32
IPW = B * F // NW
NG = 8
RPG = IPW // NG
NPLANE = 4
NSLOT = NPLANE * B * 128 // D


BPW = B // NW


def _sc_gather(x1flat, slotflat, blocflat, x2flat, table, w_flat):
    mesh = plsc.VectorSubcoreMesh(core_axis_name="c", subcore_axis_name="s")

    @functools.partial(
        pl.kernel,
        mesh=mesh,
        compiler_params=pltpu.CompilerParams(use_tc_tiling_on_sc=False, needs_layout_passes=False),
        out_type=(
            jax.ShapeDtypeStruct((NSLOT, D), jnp.float32),
            jax.ShapeDtypeStruct((B,), jnp.float32),
        ),
        scratch_types=(
            pltpu.VMEM((IPW,), jnp.int32),
            pltpu.VMEM((IPW,), jnp.int32),
            pltpu.VMEM((IPW,), jnp.int32),
            pltpu.VMEM((IPW,), jnp.float32),
            pltpu.VMEM((RPG, D), jnp.float32),
            pltpu.VMEM((RPG, D), jnp.float32),
            pltpu.VMEM((RPG,), jnp.float32),
            pltpu.VMEM((RPG,), jnp.float32),
            pltpu.VMEM((BPW,), jnp.float32),
            pltpu.SemaphoreType.DMA,
            pltpu.SemaphoreType.DMA,
            pltpu.SemaphoreType.DMA,
            pltpu.SemaphoreType.DMA,
            pltpu.SemaphoreType.DMA,
            pltpu.SemaphoreType.DMA,
        ),
    )
    def k(x1_hbm, slot_hbm, bloc_hbm, x2_hbm, tab_hbm, w_hbm, h_out, ws_out,
          idx, slot, bloc, x2v, buf0, buf1, wb0, wb1, acc,
          s0, s1, sw0, sw1, sc0, sc1):
        cid = lax.axis_index("c")
        sid = lax.axis_index("s")
        wid = sid * 2 + cid
        i0 = wid * IPW
        pltpu.sync_copy(x1_hbm.at[pl.ds(i0, IPW)], idx)
        pltpu.sync_copy(slot_hbm.at[pl.ds(i0, IPW)], slot)
        pltpu.sync_copy(bloc_hbm.at[pl.ds(i0, IPW)], bloc)
        pltpu.sync_copy(x2_hbm.at[pl.ds(i0, IPW)], x2v)
        for z in range(BPW // 16):
            acc[pl.ds(z * 16, 16)] = jnp.zeros((16,), jnp.float32)

        bufs = (buf0, buf1)
        sems = (s0, s1)
        wbufs = (wb0, wb1)
        wsems = (sw0, sw1)
        scsems = (sc0, sc1)

        def fire(g):
            return pltpu.async_copy(
                tab_hbm.at[idx.at[pl.ds(g * RPG, RPG)]],
                bufs[g % 2], sems[g % 2])

        def fire_w(g):
            return pltpu.async_copy(
                w_hbm.at[idx.at[pl.ds(g * RPG, RPG)]],
                wbufs[g % 2], wsems[g % 2])

        def fire_scatter(g):
            return pltpu.async_copy(
                bufs[g % 2], h_out.at[slot.at[pl.ds(g * RPG, RPG)]],
                scsems[g % 2])

        def accum_w(g):
            wb = wbufs[g % 2]

            def chunk(c, _):
                o = g * RPG + c * 16
                wo = c * 16
                prod = wb[pl.ds(wo, 16)] * x2v[pl.ds(o, 16)]
                plsc.addupdate_scatter(acc, [bloc[pl.ds(o, 16)]], prod)
                return _

            lax.fori_loop(0, RPG // 16, chunk, 0)

        hg = {0: fire(0)}
        hw = {0: fire_w(0)}
        hs = {}
        for g in range(NG):
            if g + 1 < NG:
                if g >= 1:
                    hs.pop(g - 1).wait()
                hg[g + 1] = fire(g + 1)
                hw[g + 1] = fire_w(g + 1)
            hg.pop(g).wait()
            hs[g] = fire_scatter(g)
            hw.pop(g).wait()
            accum_w(g)
        hs.pop(NG - 2).wait()
        hs.pop(NG - 1).wait()
        pltpu.sync_copy(acc, ws_out.at[pl.ds(wid * BPW, BPW)])

    return k(x1flat, slotflat, blocflat, x2flat, table, w_flat)


def _tc_relayout(tabT):
    CB = 8192
    G = -(-V // CB)

    def body(in_ref, out_ref):
        t = jnp.transpose(in_ref[...], (1, 0))
        t3 = t.reshape(CB // 8, 8, D)
        out_ref[...] = jnp.concatenate(
            [t3[:, j, :] for j in range(8)], axis=1)

    return pl.pallas_call(
        body,
        grid=(G,),
        in_specs=[pl.BlockSpec((D, CB), lambda i: (0, i))],
        out_specs=pl.BlockSpec((CB // 8, 128), lambda i: (i, 0)),
        out_shape=jax.ShapeDtypeStruct((V // 8, 128), jnp.float32),
    )(tabT)


def _tc_forward(h4, wsum2, x2, a11, w0p, b0p, w1p, b1p, w2p, b2p, w3, b3p):
    R = 1024
    G = B // R
    H0, H1, H2 = 100, 60, 20

    def body(h_ref, w_ref, x2_ref, a_ref, w0_ref, b0_ref, w1_ref, b1_ref,
             w2_ref, b2_ref, w3_ref, b3_ref, o_ref):
        x2b = x2_ref[...]
        s = jnp.zeros((R, D), jnp.float32)
        q = jnp.zeros((R, D), jnp.float32)
        acc = jnp.zeros((R, H0), jnp.float32)
        for j in range(NPLANE):
            hj = h_ref[j]
            for fo in range(8):
                f = j * 8 + fo
                if f >= F:
                    break
                ef = hj[:, fo * D:(fo + 1) * D] * x2b[:, f:f + 1]
                s = s + ef
                q = q + ef * ef
                acc = acc + jnp.dot(ef, w0_ref[pl.ds(f * D, D), :],
                                    preferred_element_type=jnp.float32)
        fm = 0.5 * (jnp.sum(s * s, axis=1, keepdims=True)
                    - jnp.sum(q, axis=1, keepdims=True))
        wsum = w_ref[...]
        h1 = jnp.maximum(acc + b0_ref[...], 0.0)
        h2 = jnp.maximum(jnp.dot(h1, w1_ref[...],
                                 preferred_element_type=jnp.float32)
                         + b1_ref[...], 0.0)
        h3 = jnp.maximum(jnp.dot(h2, w2_ref[...],
                                 preferred_element_type=jnp.float32)
                         + b2_ref[...], 0.0)
        deep = jnp.dot(h3, w3_ref[...],
                       preferred_element_type=jnp.float32) + b3_ref[...]
        z = (wsum + fm) * a_ref[...] + deep
        o_ref[...] = jax.nn.sigmoid(z)

    return pl.pallas_call(
        body,
        grid=(G,),
        in_specs=[
            pl.BlockSpec((NPLANE, R, 128), lambda i: (0, i, 0)),
            pl.BlockSpec((R, 1), lambda i: (i, 0)),
            pl.BlockSpec((R, F), lambda i: (i, 0)),
            pl.BlockSpec((1, 1), lambda i: (0, 0)),
            pl.BlockSpec((F * D, H0), lambda i: (0, 0)),
            pl.BlockSpec((1, H0), lambda i: (0, 0)),
            pl.BlockSpec((H0, H1), lambda i: (0, 0)),
            pl.BlockSpec((1, H1), lambda i: (0, 0)),
            pl.BlockSpec((H1, H2), lambda i: (0, 0)),
            pl.BlockSpec((1, H2), lambda i: (0, 0)),
            pl.BlockSpec((H2, 1), lambda i: (0, 0)),
            pl.BlockSpec((1, 1), lambda i: (0, 0)),
        ],
        out_specs=pl.BlockSpec((R, 1), lambda i: (i, 0)),
        out_shape=jax.ShapeDtypeStruct((B, 1), jnp.float32),
    )(h4, wsum2, x2, a11, w0p, b0p, w1p, b1p, w2p, b2p, w3, b3p)


def kernel(X1, X2, embed_table, w_table, lin_w, lin_b, w0, b0, g0, bt0,
           w1, b1, g1, bt1, w2, b2, g2, bt2, w3, b3):
    x1b = X1.reshape(-1)
    kk = jnp.arange(B * F, dtype=jnp.int32)
    bb = kk // F
    ff = kk % F
    slotc = (ff // 8) * (B * 8) + bb * 8 + (ff % 8)
    bloc = bb % BPW
    x2flat = X2.reshape(-1)
    w_flat = w_table.T.reshape(-1)
    tab_lin = _tc_relayout(embed_table.T)
    tab_sc = tab_lin.reshape(V, D)
    h_raw, wsum = _sc_gather(x1b, slotc, bloc, x2flat, tab_sc, w_flat)
    h4 = h_raw.reshape(NPLANE, B, 128)
    wsum2 = wsum[:, None]

    inv = 1.0 / jnp.sqrt(1.0 + EPS)
    s0 = g0 * inv
    s1 = g1 * inv
    s2 = g2 * inv
    w0p = w0 * s0[None, :]
    b0p = (b0 * s0 + bt0)[None, :]
    w1p = w1 * s1[None, :]
    b1p = (b1 * s1 + bt1)[None, :]
    w2p = w2 * s2[None, :]
    b2p = (b2 * s2 + bt2)[None, :]
    b3p = (b3 + lin_b)[None, :]

    return _tc_forward(h4, wsum2, X2, lin_w, w0p, b0p, w1p, b1p,
                       w2p, b2p, w3, b3p)


# speedup vs baseline: 1.2358x; 1.1300x over previous
import functools

import jax
import jax.numpy as jnp
from jax import lax
from jax.experimental import pallas as pl
from jax.experimental.pallas import tpu as pltpu
from jax.experimental.pallas import tpu_sc as plsc

B, F, V, D = 16384, 26, 1000000, 16
EPS = 1e-5

NW = 32
IPW = B * F // NW
NG = 8
RPG = IPW // NG
NPLANE = 4
NSLOT = NPLANE * B * 128 // D


BPW = B // NW


def _sc_gather(x1flat, slotflat, blocflat, x2flat, table, w_flat):
    mesh = plsc.VectorSubcoreMesh(core_axis_name="c", subcore_axis_name="s")

    @functools.partial(
        pl.kernel,
        mesh=mesh,
        compiler_params=pltpu.CompilerParams(use_tc_tiling_on_sc=False, needs_layout_passes=False),
        out_type=(
            jax.ShapeDtypeStruct((NSLOT, D), jnp.float32),
            jax.ShapeDtypeStruct((B,), jnp.float32),
        ),
        scratch_types=(
            pltpu.VMEM((IPW,), jnp.int32),
            pltpu.VMEM((IPW,), jnp.int32),
            pltpu.VMEM((IPW,), jnp.int32),
            pltpu.VMEM((IPW,), jnp.float32),
            pltpu.VMEM((RPG, D), jnp.float32),
            pltpu.VMEM((RPG, D), jnp.float32),
            pltpu.VMEM((RPG,), jnp.float32),
            pltpu.VMEM((RPG,), jnp.float32),
            pltpu.VMEM((BPW,), jnp.float32),
            pltpu.SemaphoreType.DMA,
            pltpu.SemaphoreType.DMA,
            pltpu.SemaphoreType.DMA,
            pltpu.SemaphoreType.DMA,
            pltpu.SemaphoreType.DMA,
            pltpu.SemaphoreType.DMA,
        ),
    )
    def k(x1_hbm, slot_hbm, bloc_hbm, x2_hbm, tab_hbm, w_hbm, h_out, ws_out,
          idx, slot, bloc, x2v, buf0, buf1, wb0, wb1, acc,
          s0, s1, sw0, sw1, sc0, sc1):
        cid = lax.axis_index("c")
        sid = lax.axis_index("s")
        wid = sid * 2 + cid
        i0 = wid * IPW
        pltpu.sync_copy(x1_hbm.at[pl.ds(i0, IPW)], idx)
        pltpu.sync_copy(slot_hbm.at[pl.ds(i0, IPW)], slot)
        pltpu.sync_copy(bloc_hbm.at[pl.ds(i0, IPW)], bloc)
        pltpu.sync_copy(x2_hbm.at[pl.ds(i0, IPW)], x2v)
        for z in range(BPW // 16):
            acc[pl.ds(z * 16, 16)] = jnp.zeros((16,), jnp.float32)

        bufs = (buf0, buf1)
        sems = (s0, s1)
        wbufs = (wb0, wb1)
        wsems = (sw0, sw1)
        scsems = (sc0, sc1)

        def fire(g):
            return pltpu.async_copy(
                tab_hbm.at[idx.at[pl.ds(g * RPG, RPG)]],
                bufs[g % 2], sems[g % 2])

        def fire_w(g):
            return pltpu.async_copy(
                w_hbm.at[idx.at[pl.ds(g * RPG, RPG)]],
                wbufs[g % 2], wsems[g % 2])

        def fire_scatter(g):
            return pltpu.async_copy(
                bufs[g % 2], h_out.at[slot.at[pl.ds(g * RPG, RPG)]],
                scsems[g % 2])

        def accum_w(g):
            wb = wbufs[g % 2]

            def chunk(c, _):
                o = g * RPG + c * 16
                wo = c * 16
                prod = wb[pl.ds(wo, 16)] * x2v[pl.ds(o, 16)]
                plsc.addupdate_scatter(acc, [bloc[pl.ds(o, 16)]], prod)
                return _

            lax.fori_loop(0, RPG // 16, chunk, 0)

        hg = {0: fire(0)}
        hw = {0: fire_w(0)}
        hs = {}
        for g in range(NG):
            if g + 1 < NG:
                if g >= 1:
                    hs.pop(g - 1).wait()
                hg[g + 1] = fire(g + 1)
                hw[g + 1] = fire_w(g + 1)
            hg.pop(g).wait()
            hs[g] = fire_scatter(g)
            hw.pop(g).wait()
            accum_w(g)
        hs.pop(NG - 2).wait()
        hs.pop(NG - 1).wait()
        pltpu.sync_copy(acc, ws_out.at[pl.ds(wid * BPW, BPW)])

    return k(x1flat, slotflat, blocflat, x2flat, table, w_flat)


def _tc_relayout(tabT):
    CB = 8192
    G = -(-V // CB)

    def body(in_ref, out_ref):
        t = jnp.transpose(in_ref[...], (1, 0))
        t3 = t.reshape(CB // 8, 8, D)
        out_ref[...] = jnp.concatenate(
            [t3[:, j, :] for j in range(8)], axis=1)

    return pl.pallas_call(
        body,
        grid=(G,),
        in_specs=[pl.BlockSpec((D, CB), lambda i: (0, i))],
        out_specs=pl.BlockSpec((CB // 8, 128), lambda i: (i, 0)),
        out_shape=jax.ShapeDtypeStruct((V // 8, 128), jnp.float32),
    )(tabT)


def _tc_forward(h4, wsum2, x2r4, a11, s512, w0512, b0p, w1p, b1p,
                w2p, b2p, w3, b3p):
    R = 1024
    G = B // R
    H0, H1, H2 = 100, 60, 20

    def body(h_ref, w_ref, x2r_ref, a_ref, s512_ref, w0_ref, b0_ref,
             w1_ref, b1_ref, w2_ref, b2_ref, w3_ref, b3_ref, o_ref):
        hes = []
        for j in range(NPLANE):
            x2j = x2r_ref[j]
            hes.append(jnp.where(x2j == 0.0, 0.0, h_ref[j] * x2j))
        he = jnp.concatenate(hes, axis=1)
        s = jnp.dot(he, s512_ref[...],
                    preferred_element_type=jnp.float32)
        acc = jnp.dot(he, w0_ref[...],
                      preferred_element_type=jnp.float32)
        q = jnp.sum(he * he, axis=1, keepdims=True)
        fm = 0.5 * (jnp.sum(s * s, axis=1, keepdims=True) - q)
        wsum = w_ref[...]
        h1 = jnp.maximum(acc + b0_ref[...], 0.0)
        h2 = jnp.maximum(jnp.dot(h1, w1_ref[...],
                                 preferred_element_type=jnp.float32)
                         + b1_ref[...], 0.0)
        h3 = jnp.maximum(jnp.dot(h2, w2_ref[...],
                                 preferred_element_type=jnp.float32)
                         + b2_ref[...], 0.0)
        deep = jnp.dot(h3, w3_ref[...],
                       preferred_element_type=jnp.float32) + b3_ref[...]
        z = (wsum + fm) * a_ref[...] + deep
        o_ref[...] = jax.nn.sigmoid(z)

    return pl.pallas_call(
        body,
        grid=(G,),
        in_specs=[
            pl.BlockSpec((NPLANE, R, 128), lambda i: (0, i, 0)),
            pl.BlockSpec((R, 1), lambda i: (i, 0)),
            pl.BlockSpec((NPLANE, R, 128), lambda i: (0, i, 0)),
            pl.BlockSpec((1, 1), lambda i: (0, 0)),
            pl.BlockSpec((NPLANE * 128, D), lambda i: (0, 0)),
            pl.BlockSpec((NPLANE * 128, H0), lambda i: (0, 0)),
            pl.BlockSpec((1, H0), lambda i: (0, 0)),
            pl.BlockSpec((H0, H1), lambda i: (0, 0)),
            pl.BlockSpec((1, H1), lambda i: (0, 0)),
            pl.BlockSpec((H1, H2), lambda i: (0, 0)),
            pl.BlockSpec((1, H2), lambda i: (0, 0)),
            pl.BlockSpec((H2, 1), lambda i: (0, 0)),
            pl.BlockSpec((1, 1), lambda i: (0, 0)),
        ],
        out_specs=pl.BlockSpec((R, 1), lambda i: (i, 0)),
        out_shape=jax.ShapeDtypeStruct((B, 1), jnp.float32),
    )(h4, wsum2, x2r4, a11, s512, w0512, b0p, w1p, b1p, w2p, b2p, w3, b3p)


def kernel(X1, X2, embed_table, w_table, lin_w, lin_b, w0, b0, g0, bt0,
           w1, b1, g1, bt1, w2, b2, g2, bt2, w3, b3):
    x1b = X1.reshape(-1)
    kk = jnp.arange(B * F, dtype=jnp.int32)
    bb = kk // F
    ff = kk % F
    slotc = (ff // 8) * (B * 8) + bb * 8 + (ff % 8)
    bloc = bb % BPW
    x2flat = X2.reshape(-1)
    w_flat = w_table.T.reshape(-1)
    tab_lin = _tc_relayout(embed_table.T)
    tab_sc = tab_lin.reshape(V, D)
    h_raw, wsum = _sc_gather(x1b, slotc, bloc, x2flat, tab_sc, w_flat)
    h4 = h_raw.reshape(NPLANE, B, 128)
    wsum2 = wsum[:, None]

    ll = jnp.arange(128, dtype=jnp.int32)
    planes = []
    for j in range(NPLANE):
        fj = 8 * j + ll // D
        valid = fj < F
        cols = jnp.take(X2, jnp.clip(fj, 0, F - 1), axis=1)
        planes.append(jnp.where(valid[None, :], cols, 0.0))
    x2r4 = jnp.stack(planes, axis=0)

    cc = jnp.arange(NPLANE * 128, dtype=jnp.int32)
    fc = 8 * (cc // 128) + (cc % 128) // D
    dc = cc % D
    validc = fc < F
    s512 = (validc[:, None]
            & (dc[:, None] == jnp.arange(D)[None, :])).astype(jnp.float32)

    inv = 1.0 / jnp.sqrt(1.0 + EPS)
    s0 = g0 * inv
    s1 = g1 * inv
    s2 = g2 * inv
    w0p = w0 * s0[None, :]
    w0512 = jnp.where(validc[:, None],
                      jnp.take(w0p, jnp.clip(fc * D + dc, 0, F * D - 1),
                               axis=0), 0.0)
    b0p = (b0 * s0 + bt0)[None, :]
    w1p = w1 * s1[None, :]
    b1p = (b1 * s1 + bt1)[None, :]
    w2p = w2 * s2[None, :]
    b2p = (b2 * s2 + bt2)[None, :]
    b3p = (b3 + lin_b)[None, :]

    return _tc_forward(h4, wsum2, x2r4, lin_w, s512, w0512, b0p, w1p, b1p,
                       w2p, b2p, w3, b3p)


# speedup vs baseline: 1.2391x; 1.0027x over previous
import functools

import jax
import jax.numpy as jnp
from jax import lax
from jax.experimental import pallas as pl
from jax.experimental.pallas import tpu as pltpu
from jax.experimental.pallas import tpu_sc as plsc

B, F, V, D = 16384, 26, 1000000, 16
EPS = 1e-5

NW = 32
IPW = B * F // NW
NG = 8
RPG = IPW // NG
NPLANE = 4
NSLOT = NPLANE * B * 128 // D


BPW = B // NW


def _sc_gather(x1flat, slotflat, blocflat, x2flat, table, w_flat):
    mesh = plsc.VectorSubcoreMesh(core_axis_name="c", subcore_axis_name="s")

    @functools.partial(
        pl.kernel,
        mesh=mesh,
        compiler_params=pltpu.CompilerParams(use_tc_tiling_on_sc=False, needs_layout_passes=False),
        out_type=(
            jax.ShapeDtypeStruct((NSLOT, D), jnp.float32),
            jax.ShapeDtypeStruct((B,), jnp.float32),
        ),
        scratch_types=(
            pltpu.VMEM((IPW,), jnp.int32),
            pltpu.VMEM((IPW,), jnp.int32),
            pltpu.VMEM((IPW,), jnp.int32),
            pltpu.VMEM((IPW,), jnp.float32),
            pltpu.VMEM((RPG, D), jnp.float32),
            pltpu.VMEM((RPG, D), jnp.float32),
            pltpu.VMEM((RPG,), jnp.float32),
            pltpu.VMEM((RPG,), jnp.float32),
            pltpu.VMEM((BPW,), jnp.float32),
            pltpu.SemaphoreType.DMA,
            pltpu.SemaphoreType.DMA,
            pltpu.SemaphoreType.DMA,
            pltpu.SemaphoreType.DMA,
            pltpu.SemaphoreType.DMA,
            pltpu.SemaphoreType.DMA,
        ),
    )
    def k(x1_hbm, slot_hbm, bloc_hbm, x2_hbm, tab_hbm, w_hbm, h_out, ws_out,
          idx, slot, bloc, x2v, buf0, buf1, wb0, wb1, acc,
          s0, s1, sw0, sw1, sc0, sc1):
        cid = lax.axis_index("c")
        sid = lax.axis_index("s")
        wid = sid * 2 + cid
        i0 = wid * IPW
        pltpu.sync_copy(x1_hbm.at[pl.ds(i0, IPW)], idx)
        pltpu.sync_copy(slot_hbm.at[pl.ds(i0, IPW)], slot)
        pltpu.sync_copy(bloc_hbm.at[pl.ds(i0, IPW)], bloc)
        pltpu.sync_copy(x2_hbm.at[pl.ds(i0, IPW)], x2v)
        for z in range(BPW // 16):
            acc[pl.ds(z * 16, 16)] = jnp.zeros((16,), jnp.float32)

        bufs = (buf0, buf1)
        sems = (s0, s1)
        wbufs = (wb0, wb1)
        wsems = (sw0, sw1)
        scsems = (sc0, sc1)

        def fire(g):
            return pltpu.async_copy(
                tab_hbm.at[idx.at[pl.ds(g * RPG, RPG)]],
                bufs[g % 2], sems[g % 2])

        def fire_w(g):
            return pltpu.async_copy(
                w_hbm.at[idx.at[pl.ds(g * RPG, RPG)]],
                wbufs[g % 2], wsems[g % 2])

        def fire_scatter(g):
            return pltpu.async_copy(
                bufs[g % 2], h_out.at[slot.at[pl.ds(g * RPG, RPG)]],
                scsems[g % 2])

        def accum_w(g):
            wb = wbufs[g % 2]

            def chunk(c, _):
                o = g * RPG + c * 16
                wo = c * 16
                prod = wb[pl.ds(wo, 16)] * x2v[pl.ds(o, 16)]
                plsc.addupdate_scatter(acc, [bloc[pl.ds(o, 16)]], prod)
                return _

            lax.fori_loop(0, RPG // 16, chunk, 0)

        hg = {0: fire(0)}
        hw = {0: fire_w(0)}
        hs = {}
        for g in range(NG):
            if g + 1 < NG:
                if g >= 1:
                    hs.pop(g - 1).wait()
                hg[g + 1] = fire(g + 1)
                hw[g + 1] = fire_w(g + 1)
            hg.pop(g).wait()
            hs[g] = fire_scatter(g)
            hw.pop(g).wait()
            accum_w(g)
        hs.pop(NG - 2).wait()
        hs.pop(NG - 1).wait()
        pltpu.sync_copy(acc, ws_out.at[pl.ds(wid * BPW, BPW)])

    return k(x1flat, slotflat, blocflat, x2flat, table, w_flat)


def _tc_relayout(tabT):
    CB = 8192
    G = -(-V // CB)

    def body(in_ref, eye_ref, sel_ref, out_ref):
        t = jax.lax.dot_general(in_ref[...], eye_ref[...],
                                (((0,), (0,)), ((), ())),
                                preferred_element_type=jnp.float32)
        t3 = t.reshape(CB // 8, 8, D)
        acc = jnp.zeros((CB // 8, 128), jnp.float32)
        for j in range(8):
            acc = acc + jnp.dot(t3[:, j, :],
                                sel_ref[pl.ds(j * D, D), :],
                                preferred_element_type=jnp.float32)
        out_ref[...] = acc

    eye = jnp.eye(D, dtype=jnp.float32)
    sel = jnp.zeros((8 * D, 128), jnp.float32)
    rr = jnp.arange(8 * D)
    sel = sel.at[rr, rr].set(1.0)
    return pl.pallas_call(
        body,
        grid=(G,),
        in_specs=[
            pl.BlockSpec((D, CB), lambda i: (0, i)),
            pl.BlockSpec((D, D), lambda i: (0, 0)),
            pl.BlockSpec((8 * D, 128), lambda i: (0, 0)),
        ],
        out_specs=pl.BlockSpec((CB // 8, 128), lambda i: (i, 0)),
        out_shape=jax.ShapeDtypeStruct((V // 8, 128), jnp.float32),
    )(tabT, eye, sel)


def _tc_forward(h4, wsum2, x2r4, a11, s512, w0512, b0p, w1p, b1p,
                w2p, b2p, w3, b3p):
    R = 1024
    G = B // R
    H0, H1, H2 = 100, 60, 20

    def body(h_ref, w_ref, x2r_ref, a_ref, s512_ref, w0_ref, b0_ref,
             w1_ref, b1_ref, w2_ref, b2_ref, w3_ref, b3_ref, o_ref):
        hes = []
        for j in range(NPLANE):
            x2j = x2r_ref[j]
            hes.append(jnp.where(x2j == 0.0, 0.0, h_ref[j] * x2j))
        he = jnp.concatenate(hes, axis=1)
        s = jnp.dot(he, s512_ref[...],
                    preferred_element_type=jnp.float32)
        acc = jnp.dot(he, w0_ref[...],
                      preferred_element_type=jnp.float32)
        q = jnp.sum(he * he, axis=1, keepdims=True)
        fm = 0.5 * (jnp.sum(s * s, axis=1, keepdims=True) - q)
        wsum = w_ref[...]
        h1 = jnp.maximum(acc + b0_ref[...], 0.0)
        h2 = jnp.maximum(jnp.dot(h1, w1_ref[...],
                                 preferred_element_type=jnp.float32)
                         + b1_ref[...], 0.0)
        h3 = jnp.maximum(jnp.dot(h2, w2_ref[...],
                                 preferred_element_type=jnp.float32)
                         + b2_ref[...], 0.0)
        deep = jnp.dot(h3, w3_ref[...],
                       preferred_element_type=jnp.float32) + b3_ref[...]
        z = (wsum + fm) * a_ref[...] + deep
        o_ref[...] = jax.nn.sigmoid(z)

    return pl.pallas_call(
        body,
        grid=(G,),
        in_specs=[
            pl.BlockSpec((NPLANE, R, 128), lambda i: (0, i, 0)),
            pl.BlockSpec((R, 1), lambda i: (i, 0)),
            pl.BlockSpec((NPLANE, R, 128), lambda i: (0, i, 0)),
            pl.BlockSpec((1, 1), lambda i: (0, 0)),
            pl.BlockSpec((NPLANE * 128, D), lambda i: (0, 0)),
            pl.BlockSpec((NPLANE * 128, H0), lambda i: (0, 0)),
            pl.BlockSpec((1, H0), lambda i: (0, 0)),
            pl.BlockSpec((H0, H1), lambda i: (0, 0)),
            pl.BlockSpec((1, H1), lambda i: (0, 0)),
            pl.BlockSpec((H1, H2), lambda i: (0, 0)),
            pl.BlockSpec((1, H2), lambda i: (0, 0)),
            pl.BlockSpec((H2, 1), lambda i: (0, 0)),
            pl.BlockSpec((1, 1), lambda i: (0, 0)),
        ],
        out_specs=pl.BlockSpec((R, 1), lambda i: (i, 0)),
        out_shape=jax.ShapeDtypeStruct((B, 1), jnp.float32),
    )(h4, wsum2, x2r4, a11, s512, w0512, b0p, w1p, b1p, w2p, b2p, w3, b3p)


def kernel(X1, X2, embed_table, w_table, lin_w, lin_b, w0, b0, g0, bt0,
           w1, b1, g1, bt1, w2, b2, g2, bt2, w3, b3):
    x1b = X1.reshape(-1)
    kk = jnp.arange(B * F, dtype=jnp.int32)
    bb = kk // F
    ff = kk % F
    slotc = (ff // 8) * (B * 8) + bb * 8 + (ff % 8)
    bloc = bb % BPW
    x2flat = X2.reshape(-1)
    w_flat = w_table.T.reshape(-1)
    tab_lin = _tc_relayout(embed_table.T)
    tab_sc = tab_lin.reshape(V, D)
    h_raw, wsum = _sc_gather(x1b, slotc, bloc, x2flat, tab_sc, w_flat)
    h4 = h_raw.reshape(NPLANE, B, 128)
    wsum2 = wsum[:, None]

    ll = jnp.arange(128, dtype=jnp.int32)
    planes = []
    for j in range(NPLANE):
        fj = 8 * j + ll // D
        valid = fj < F
        cols = jnp.take(X2, jnp.clip(fj, 0, F - 1), axis=1)
        planes.append(jnp.where(valid[None, :], cols, 0.0))
    x2r4 = jnp.stack(planes, axis=0)

    cc = jnp.arange(NPLANE * 128, dtype=jnp.int32)
    fc = 8 * (cc // 128) + (cc % 128) // D
    dc = cc % D
    validc = fc < F
    s512 = (validc[:, None]
            & (dc[:, None] == jnp.arange(D)[None, :])).astype(jnp.float32)

    inv = 1.0 / jnp.sqrt(1.0 + EPS)
    s0 = g0 * inv
    s1 = g1 * inv
    s2 = g2 * inv
    w0p = w0 * s0[None, :]
    w0512 = jnp.where(validc[:, None],
                      jnp.take(w0p, jnp.clip(fc * D + dc, 0, F * D - 1),
                               axis=0), 0.0)
    b0p = (b0 * s0 + bt0)[None, :]
    w1p = w1 * s1[None, :]
    b1p = (b1 * s1 + bt1)[None, :]
    w2p = w2 * s2[None, :]
    b2p = (b2 * s2 + bt2)[None, :]
    b3p = (b3 + lin_b)[None, :]

    return _tc_forward(h4, wsum2, x2r4, lin_w, s512, w0512, b0p, w1p, b1p,
                       w2p, b2p, w3, b3p)


# speedup vs baseline: 1.2724x; 1.0269x over previous
import functools

import jax
import jax.numpy as jnp
from jax import lax
from jax.experimental import pallas as pl
from jax.experimental.pallas import tpu as pltpu
from jax.experimental.pallas import tpu_sc as plsc

B, F, V, D = 16384, 26, 1000000, 16
EPS = 1e-5

NW = 32
IPW = B * F // NW
NG = 8
RPG = IPW // NG
NPLANE = 4
NSLOT = NPLANE * B * 128 // D


BPW = B // NW


def _sc_gather(x1flat, slotflat, blocflat, x2flat, table, w_flat):
    mesh = plsc.VectorSubcoreMesh(core_axis_name="c", subcore_axis_name="s")

    @functools.partial(
        pl.kernel,
        mesh=mesh,
        compiler_params=pltpu.CompilerParams(use_tc_tiling_on_sc=False, needs_layout_passes=False),
        out_type=(
            jax.ShapeDtypeStruct((NSLOT, D), jnp.float32),
            jax.ShapeDtypeStruct((B,), jnp.float32),
        ),
        scratch_types=(
            pltpu.VMEM((IPW,), jnp.int32),
            pltpu.VMEM((IPW,), jnp.int32),
            pltpu.VMEM((IPW,), jnp.int32),
            pltpu.VMEM((IPW,), jnp.float32),
            pltpu.VMEM((RPG, D), jnp.float32),
            pltpu.VMEM((RPG, D), jnp.float32),
            pltpu.VMEM((RPG,), jnp.float32),
            pltpu.VMEM((RPG,), jnp.float32),
            pltpu.VMEM((BPW,), jnp.float32),
            pltpu.SemaphoreType.DMA,
            pltpu.SemaphoreType.DMA,
            pltpu.SemaphoreType.DMA,
            pltpu.SemaphoreType.DMA,
            pltpu.SemaphoreType.DMA,
            pltpu.SemaphoreType.DMA,
        ),
    )
    def k(x1_hbm, slot_hbm, bloc_hbm, x2_hbm, tab_hbm, w_hbm, h_out, ws_out,
          idx, slot, bloc, x2v, buf0, buf1, wb0, wb1, acc,
          s0, s1, sw0, sw1, sc0, sc1):
        cid = lax.axis_index("c")
        sid = lax.axis_index("s")
        wid = sid * 2 + cid
        i0 = wid * IPW
        pltpu.sync_copy(x1_hbm.at[pl.ds(i0, IPW)], idx)
        pltpu.sync_copy(slot_hbm.at[pl.ds(i0, IPW)], slot)
        pltpu.sync_copy(bloc_hbm.at[pl.ds(i0, IPW)], bloc)
        pltpu.sync_copy(x2_hbm.at[pl.ds(i0, IPW)], x2v)
        for z in range(BPW // 16):
            acc[pl.ds(z * 16, 16)] = jnp.zeros((16,), jnp.float32)

        bufs = (buf0, buf1)
        sems = (s0, s1)
        wbufs = (wb0, wb1)
        wsems = (sw0, sw1)
        scsems = (sc0, sc1)

        def fire(g):
            return pltpu.async_copy(
                tab_hbm.at[idx.at[pl.ds(g * RPG, RPG)]],
                bufs[g % 2], sems[g % 2])

        def fire_w(g):
            return pltpu.async_copy(
                w_hbm.at[idx.at[pl.ds(g * RPG, RPG)]],
                wbufs[g % 2], wsems[g % 2])

        def fire_scatter(g):
            return pltpu.async_copy(
                bufs[g % 2], h_out.at[slot.at[pl.ds(g * RPG, RPG)]],
                scsems[g % 2])

        def accum_w(g):
            wb = wbufs[g % 2]

            def chunk(c, _):
                o = g * RPG + c * 16
                wo = c * 16
                prod = wb[pl.ds(wo, 16)] * x2v[pl.ds(o, 16)]
                plsc.addupdate_scatter(acc, [bloc[pl.ds(o, 16)]], prod)
                return _

            lax.fori_loop(0, RPG // 16, chunk, 0)

        hg = {0: fire(0)}
        hw = {0: fire_w(0)}
        hs = {}
        for g in range(NG):
            if g + 1 < NG:
                if g >= 1:
                    hs.pop(g - 1).wait()
                hg[g + 1] = fire(g + 1)
                hw[g + 1] = fire_w(g + 1)
            hg.pop(g).wait()
            hs[g] = fire_scatter(g)
            hw.pop(g).wait()
            accum_w(g)
        hs.pop(NG - 2).wait()
        hs.pop(NG - 1).wait()
        pltpu.sync_copy(acc, ws_out.at[pl.ds(wid * BPW, BPW)])

    return k(x1flat, slotflat, blocflat, x2flat, table, w_flat)


def _tc_relayout(tabT):
    CB = 32768
    G = -(-V // CB)

    def body(in_ref, eye_ref, sel_ref, out_ref):
        t = jax.lax.dot_general(in_ref[...], eye_ref[...],
                                (((0,), (0,)), ((), ())),
                                preferred_element_type=jnp.float32)
        t3 = t.reshape(CB // 8, 8, D)
        acc = jnp.zeros((CB // 8, 128), jnp.float32)
        for j in range(8):
            acc = acc + jnp.dot(t3[:, j, :],
                                sel_ref[pl.ds(j * D, D), :],
                                preferred_element_type=jnp.float32)
        out_ref[...] = acc

    eye = jnp.eye(D, dtype=jnp.float32)
    sel = jnp.zeros((8 * D, 128), jnp.float32)
    rr = jnp.arange(8 * D)
    sel = sel.at[rr, rr].set(1.0)
    return pl.pallas_call(
        body,
        grid=(G,),
        in_specs=[
            pl.BlockSpec((D, CB), lambda i: (0, i)),
            pl.BlockSpec((D, D), lambda i: (0, 0)),
            pl.BlockSpec((8 * D, 128), lambda i: (0, 0)),
        ],
        out_specs=pl.BlockSpec((CB // 8, 128), lambda i: (i, 0)),
        out_shape=jax.ShapeDtypeStruct((V // 8, 128), jnp.float32),
    )(tabT, eye, sel)


def _tc_forward(h4, wsum2, x2r4, a11, s512, w0512, b0p, w1p, b1p,
                w2p, b2p, w3, b3p):
    R = 1024
    G = B // R
    H0, H1, H2 = 100, 60, 20

    def body(h_ref, w_ref, x2r_ref, a_ref, s512_ref, w0_ref, b0_ref,
             w1_ref, b1_ref, w2_ref, b2_ref, w3_ref, b3_ref, o_ref):
        hes = []
        for j in range(NPLANE):
            x2j = x2r_ref[j]
            hes.append(jnp.where(x2j == 0.0, 0.0, h_ref[j] * x2j))
        he = jnp.concatenate(hes, axis=1)
        s = jnp.dot(he, s512_ref[...],
                    preferred_element_type=jnp.float32)
        acc = jnp.dot(he, w0_ref[...],
                      preferred_element_type=jnp.float32)
        q = jnp.sum(he * he, axis=1, keepdims=True)
        fm = 0.5 * (jnp.sum(s * s, axis=1, keepdims=True) - q)
        wsum = w_ref[...]
        h1 = jnp.maximum(acc + b0_ref[...], 0.0)
        h2 = jnp.maximum(jnp.dot(h1, w1_ref[...],
                                 preferred_element_type=jnp.float32)
                         + b1_ref[...], 0.0)
        h3 = jnp.maximum(jnp.dot(h2, w2_ref[...],
                                 preferred_element_type=jnp.float32)
                         + b2_ref[...], 0.0)
        deep = jnp.dot(h3, w3_ref[...],
                       preferred_element_type=jnp.float32) + b3_ref[...]
        z = (wsum + fm) * a_ref[...] + deep
        o_ref[...] = jax.nn.sigmoid(z)

    return pl.pallas_call(
        body,
        grid=(G,),
        in_specs=[
            pl.BlockSpec((NPLANE, R, 128), lambda i: (0, i, 0)),
            pl.BlockSpec((R, 1), lambda i: (i, 0)),
            pl.BlockSpec((NPLANE, R, 128), lambda i: (0, i, 0)),
            pl.BlockSpec((1, 1), lambda i: (0, 0)),
            pl.BlockSpec((NPLANE * 128, D), lambda i: (0, 0)),
            pl.BlockSpec((NPLANE * 128, H0), lambda i: (0, 0)),
            pl.BlockSpec((1, H0), lambda i: (0, 0)),
            pl.BlockSpec((H0, H1), lambda i: (0, 0)),
            pl.BlockSpec((1, H1), lambda i: (0, 0)),
            pl.BlockSpec((H1, H2), lambda i: (0, 0)),
            pl.BlockSpec((1, H2), lambda i: (0, 0)),
            pl.BlockSpec((H2, 1), lambda i: (0, 0)),
            pl.BlockSpec((1, 1), lambda i: (0, 0)),
        ],
        out_specs=pl.BlockSpec((R, 1), lambda i: (i, 0)),
        out_shape=jax.ShapeDtypeStruct((B, 1), jnp.float32),
    )(h4, wsum2, x2r4, a11, s512, w0512, b0p, w1p, b1p, w2p, b2p, w3, b3p)


def kernel(X1, X2, embed_table, w_table, lin_w, lin_b, w0, b0, g0, bt0,
           w1, b1, g1, bt1, w2, b2, g2, bt2, w3, b3):
    x1b = X1.reshape(-1)
    kk = jnp.arange(B * F, dtype=jnp.int32)
    bb = kk // F
    ff = kk % F
    slotc = (ff // 8) * (B * 8) + bb * 8 + (ff % 8)
    bloc = bb % BPW
    x2flat = X2.reshape(-1)
    w_flat = w_table.T.reshape(-1)
    tab_lin = _tc_relayout(embed_table.T)
    tab_sc = tab_lin.reshape(V, D)
    h_raw, wsum = _sc_gather(x1b, slotc, bloc, x2flat, tab_sc, w_flat)
    h4 = h_raw.reshape(NPLANE, B, 128)
    wsum2 = wsum[:, None]

    ll = jnp.arange(128, dtype=jnp.int32)
    planes = []
    for j in range(NPLANE):
        fj = 8 * j + ll // D
        valid = fj < F
        cols = jnp.take(X2, jnp.clip(fj, 0, F - 1), axis=1)
        planes.append(jnp.where(valid[None, :], cols, 0.0))
    x2r4 = jnp.stack(planes, axis=0)

    cc = jnp.arange(NPLANE * 128, dtype=jnp.int32)
    fc = 8 * (cc // 128) + (cc % 128) // D
    dc = cc % D
    validc = fc < F
    s512 = (validc[:, None]
            & (dc[:, None] == jnp.arange(D)[None, :])).astype(jnp.float32)

    inv = 1.0 / jnp.sqrt(1.0 + EPS)
    s0 = g0 * inv
    s1 = g1 * inv
    s2 = g2 * inv
    w0p = w0 * s0[None, :]
    w0512 = jnp.where(validc[:, None],
                      jnp.take(w0p, jnp.clip(fc * D + dc, 0, F * D - 1),
                               axis=0), 0.0)
    b0p = (b0 * s0 + bt0)[None, :]
    w1p = w1 * s1[None, :]
    b1p = (b1 * s1 + bt1)[None, :]
    w2p = w2 * s2[None, :]
    b2p = (b2 * s2 + bt2)[None, :]
    b3p = (b3 + lin_b)[None, :]

    return _tc_forward(h4, wsum2, x2r4, lin_w, s512, w0512, b0p, w1p, b1p,
                       w2p, b2p, w3, b3p)
